# baseline (speedup 1.0000x reference)
import functools
import math

import jax
import jax.numpy as jnp
from jax.experimental import pallas as pl
from jax.experimental.pallas import tpu as pltpu

N = 10000
E = 320000
D = 128
HEADS = 6
RATIO = 0.5
NEG_SLOPE = 0.2


def _segment_softmax(logits, seg, num_segs):
    m = jax.ops.segment_max(logits, seg, num_segments=num_segs)
    m = jnp.where(jnp.isfinite(m), m, 0.0)
    e = jnp.exp(logits - m[seg])
    s = jax.ops.segment_sum(e, seg, num_segments=num_segs)
    return e / (s[seg] + 1e-16)


def _gather_scale_body(perm_ref, ts_ref, xc_ref, out_ref):
    out_ref[...] = xc_ref[...] * ts_ref[...]


def _new_x_pallas(x_cluster, top_scores, perm):
    k = top_scores.shape[0]
    ts_b = jnp.broadcast_to(top_scores[:, None, None], (k, 1, D))
    xc3 = x_cluster[:, None, :]
    grid_spec = pltpu.PrefetchScalarGridSpec(
        num_scalar_prefetch=1,
        grid=(k,),
        in_specs=[
            pl.BlockSpec((1, 1, D), lambda i, perm_ref: (i, 0, 0)),
            pl.BlockSpec((1, 1, D), lambda i, perm_ref: (perm_ref[i], 0, 0)),
        ],
        out_specs=pl.BlockSpec((1, 1, D), lambda i, perm_ref: (i, 0, 0)),
    )
    out = pl.pallas_call(
        _gather_scale_body,
        grid_spec=grid_spec,
        out_shape=jax.ShapeDtypeStruct((k, 1, D), jnp.float32),
    )(perm, ts_b, xc3)
    return out[:, 0, :]


def kernel(x, edge_index, W_gnn, b_gnn, weight, attention, lin_W, lin_b, s_lin1_W, s_lin2_W, s_lin2_b):
    loop = jnp.arange(N, dtype=edge_index.dtype)
    src = jnp.concatenate([edge_index[0], loop])
    dst = jnp.concatenate([edge_index[1], loop])
    ones = jnp.ones_like(src, jnp.float32)
    deg = jax.ops.segment_sum(ones, dst, num_segments=N)
    dinv = 1.0 / jnp.sqrt(jnp.clip(deg, 1.0, None))
    h = x @ W_gnn
    norm = dinv[src] * dinv[dst]
    x_pool_j = jax.ops.segment_sum(norm[:, None] * h[src], dst, num_segments=N) + b_gnn
    xpj = (x_pool_j @ weight).reshape(N, HEADS, D)
    x_q = x @ lin_W + lin_b
    q_i = jnp.broadcast_to(x_q[dst][:, None, :], (src.shape[0], HEADS, D))
    cat = jnp.concatenate([q_i, xpj[src]], axis=-1)
    logits = jnp.sum(cat * attention, axis=-1)
    logits = jax.nn.leaky_relu(logits, NEG_SLOPE)
    alpha = _segment_softmax(logits, dst, N)
    x_cluster_h = jax.ops.segment_sum(alpha[:, :, None] * xpj[src], dst, num_segments=N)
    x_cluster = jnp.mean(x_cluster_h, axis=1)
    a = x_cluster @ s_lin1_W
    b = x_cluster @ s_lin2_W + s_lin2_b
    ai, aj = a[dst], a[src]
    cos = jnp.sum(ai * aj, axis=-1) / (jnp.linalg.norm(ai, axis=-1) * jnp.linalg.norm(aj, axis=-1) + 1e-8)
    fitness = jax.ops.segment_sum(b[dst] * cos[:, None], dst, num_segments=N)
    score = jax.nn.sigmoid(fitness[:, 0])
    k = int(math.ceil(RATIO * N))
    top_scores, perm = jax.lax.top_k(score, k)
    new_x = _new_x_pallas(x_cluster, top_scores, perm)
    return new_x, score, perm


# speedup vs baseline: 2.2672x; 2.2672x over previous
import functools
import math

import jax
import jax.numpy as jnp
from jax import lax
from jax.experimental import pallas as pl
from jax.experimental.pallas import tpu as pltpu
from jax.experimental.pallas import tpu_sc as plsc

N = 10000
E = 320000
D = 128
HEADS = 6
RATIO = 0.5
NEG_SLOPE = 0.2

NW = 32
P = 320
NPAD = NW * P
EP = E + N
EPAD = 330240
K = 128
RPLEN = NPAD + 16


def _segment_softmax(logits, seg, num_segs):
    m = jax.ops.segment_max(logits, seg, num_segments=num_segs)
    m = jnp.where(jnp.isfinite(m), m, 0.0)
    e = jnp.exp(logits - m[seg])
    s = jax.ops.segment_sum(e, seg, num_segments=num_segs)
    return e / (s[seg] + 1e-16)


def _fold_body(xpjr, srcs, dsts, alphf, rp, out, srcs_v, gidx_v, dsts_v,
               alph_v, rows_v, acc, rp_v, tmp_v, sem):
    nc = 2
    wid = lax.axis_index("s") * nc + lax.axis_index("c")
    lo = wid * P
    pltpu.sync_copy(rp.at[pl.ds(lo, 16)], rp_v)
    e0 = rp_v[...][0]
    pltpu.sync_copy(rp.at[pl.ds(lo + P, 16)], rp_v)
    e1 = rp_v[...][0]
    a0 = (e0 // 8) * 8
    nb = (e1 - a0 + K - 1) // K

    zeros16 = jnp.zeros((16,), jnp.float32)

    for h in range(HEADS):
        def zrow(r, _):
            for q in range(8):
                acc[r, pl.ds(q * 16, 16)] = zeros16
            return 0

        lax.fori_loop(0, P, zrow, 0)

        def batch(i, _):
            base = a0 + i * K
            pltpu.sync_copy(srcs.at[pl.ds(base, K)], srcs_v)
            pltpu.sync_copy(dsts.at[pl.ds(base, K)], dsts_v)
            pltpu.sync_copy(alphf.at[pl.ds(h * EPAD + base, K)], alph_v)

            def gix(c, _):
                sl = pl.ds(c * 16, 16)
                gidx_v[sl] = srcs_v[sl] + h * N
                return 0

            lax.fori_loop(0, K // 16, gix, 0)
            pltpu.async_copy(xpjr.at[gidx_v], rows_v, sem).wait()

            def chunk(c, _):
                dv = dsts_v[pl.ds(c * 16, 16)]
                av = alph_v[pl.ds(c * 16, 16)]
                for l in range(16):
                    pos = base + c * 16 + l
                    valid = jnp.logical_and(pos >= e0, pos < e1)
                    d_s = dv[l] - lo
                    d_s = jnp.where(valid, d_s, 0)
                    d_s = jnp.clip(d_s, 0, P - 1)
                    a_s = av[l] * valid.astype(jnp.float32)
                    for q in range(8):
                        sl = pl.ds(q * 16, 16)
                        tmp_v[pl.ds(q * 16, 16)] = a_s * rows_v[c * 16 + l, sl]
                    for q in range(8):
                        sl = pl.ds(q * 16, 16)
                        acc[d_s, sl] = acc[d_s, sl] + tmp_v[sl]
                return 0

            lax.fori_loop(0, K // 16, chunk, 0)
            return 0

        lax.fori_loop(0, nb, batch, 0)

        @pl.when(lo + P <= N)
        def _():
            pltpu.sync_copy(acc, out.at[pl.ds(lo, P), h])

        @pl.when(lo + P > N)
        def _():
            pltpu.sync_copy(acc.at[pl.ds(0, N - (NW - 1) * P)],
                            out.at[pl.ds(lo, N - (NW - 1) * P), h])


@jax.jit
def _fold_sc(xpjr, srcs, dsts, alphf, rp):
    mesh = plsc.VectorSubcoreMesh(core_axis_name="c", subcore_axis_name="s")
    fn = pl.kernel(
        _fold_body,
        mesh=mesh,
        out_type=jax.ShapeDtypeStruct((N, HEADS, D), jnp.float32),
        scratch_types=[
            pltpu.VMEM((K,), jnp.int32),
            pltpu.VMEM((K,), jnp.int32),
            pltpu.VMEM((K,), jnp.int32),
            pltpu.VMEM((K,), jnp.float32),
            pltpu.VMEM((K, D), jnp.float32),
            pltpu.VMEM((P, D), jnp.float32),
            pltpu.VMEM((16,), jnp.int32),
            pltpu.VMEM((D,), jnp.float32),
            pltpu.SemaphoreType.DMA,
        ],
    )
    return fn(xpjr, srcs, dsts, alphf, rp)


def kernel(x, edge_index, W_gnn, b_gnn, weight, attention, lin_W, lin_b, s_lin1_W, s_lin2_W, s_lin2_b):
    loop = jnp.arange(N, dtype=edge_index.dtype)
    src = jnp.concatenate([edge_index[0], loop])
    dst = jnp.concatenate([edge_index[1], loop])
    ones = jnp.ones_like(src, jnp.float32)
    deg = jax.ops.segment_sum(ones, dst, num_segments=N)
    dinv = 1.0 / jnp.sqrt(jnp.clip(deg, 1.0, None))
    h = x @ W_gnn
    norm = dinv[src] * dinv[dst]
    x_pool_j = jax.ops.segment_sum(norm[:, None] * h[src], dst, num_segments=N) + b_gnn
    xpj = (x_pool_j @ weight).reshape(N, HEADS, D)
    x_q = x @ lin_W + lin_b
    q_i = jnp.broadcast_to(x_q[dst][:, None, :], (src.shape[0], HEADS, D))
    cat = jnp.concatenate([q_i, xpj[src]], axis=-1)
    logits = jnp.sum(cat * attention, axis=-1)
    logits = jax.nn.leaky_relu(logits, NEG_SLOPE)
    alpha = _segment_softmax(logits, dst, N)

    order = jnp.argsort(dst, stable=True).astype(jnp.int32)
    dsts_s = dst[order].astype(jnp.int32)
    srcs_s = src[order].astype(jnp.int32)
    degp = jax.ops.segment_sum(jnp.ones((EP,), jnp.int32), dst, num_segments=NPAD)
    rp = jnp.concatenate([jnp.zeros((1,), jnp.int32), jnp.cumsum(degp).astype(jnp.int32)])
    rp = jnp.concatenate([rp, jnp.full((RPLEN - rp.shape[0],), EP, jnp.int32)])
    srcs_pad = jnp.concatenate([srcs_s, jnp.zeros((EPAD - EP,), jnp.int32)])
    dsts_pad = jnp.concatenate([dsts_s, jnp.zeros((EPAD - EP,), jnp.int32)])
    alph_s = alpha[order]
    alphf = jnp.concatenate(
        [alph_s.T, jnp.zeros((HEADS, EPAD - EP), jnp.float32)], axis=1
    ).reshape(HEADS * EPAD)
    xpjr = xpj.transpose(1, 0, 2).reshape(HEADS * N, D)
    x_cluster_h = _fold_sc(xpjr, srcs_pad, dsts_pad, alphf, rp)
    x_cluster = jnp.mean(x_cluster_h, axis=1)
    a = x_cluster @ s_lin1_W
    b = x_cluster @ s_lin2_W + s_lin2_b
    ai, aj = a[dst], a[src]
    cos = jnp.sum(ai * aj, axis=-1) / (jnp.linalg.norm(ai, axis=-1) * jnp.linalg.norm(aj, axis=-1) + 1e-8)
    fitness = jax.ops.segment_sum(b[dst] * cos[:, None], dst, num_segments=N)
    score = jax.nn.sigmoid(fitness[:, 0])
    k = int(math.ceil(RATIO * N))
    top_scores, perm = jax.lax.top_k(score, k)
    new_x = x_cluster[perm] * top_scores[:, None]
    return new_x, score, perm


# speedup vs baseline: 2.2673x; 1.0000x over previous
import math

import jax
import jax.numpy as jnp
from jax import lax
from jax.experimental import pallas as pl
from jax.experimental.pallas import tpu as pltpu
from jax.experimental.pallas import tpu_sc as plsc

N = 10000
E = 320000
D = 128
HEADS = 6
RATIO = 0.5
NEG_SLOPE = 0.2

NW = 32
P = 320
NPAD = NW * P
EP = E + N
EPAD = 330240
K = 128
RPLEN = NPAD + 16


def _segment_softmax(logits, seg, num_segs):
    m = jax.ops.segment_max(logits, seg, num_segments=num_segs)
    m = jnp.where(jnp.isfinite(m), m, 0.0)
    e = jnp.exp(logits - m[seg])
    s = jax.ops.segment_sum(e, seg, num_segments=num_segs)
    return e / (s[seg] + 1e-16)


def _fold_body(xpjr, srcs, dsts, alphf, rp, out, srcs_v, gidx_v, dsts_v,
               alph_v, rows_v, acc, rp_v, tmp_v, sem):
    nc = 2
    wid = lax.axis_index("s") * nc + lax.axis_index("c")
    lo = wid * P
    pltpu.sync_copy(rp.at[pl.ds(lo, 16)], rp_v)
    e0 = rp_v[...][0]
    pltpu.sync_copy(rp.at[pl.ds(lo + P, 16)], rp_v)
    e1 = rp_v[...][0]
    a0 = (e0 // 8) * 8
    nb = (e1 - a0 + K - 1) // K

    zeros16 = jnp.zeros((16,), jnp.float32)

    for h in range(HEADS):
        def zrow(r, _):
            for q in range(8):
                acc[r, pl.ds(q * 16, 16)] = zeros16
            return 0

        lax.fori_loop(0, P, zrow, 0)

        def batch(i, _):
            base = a0 + i * K
            pltpu.sync_copy(srcs.at[pl.ds(base, K)], srcs_v)
            pltpu.sync_copy(dsts.at[pl.ds(base, K)], dsts_v)
            pltpu.sync_copy(alphf.at[pl.ds(h * EPAD + base, K)], alph_v)

            def gix(c, _):
                sl = pl.ds(c * 16, 16)
                gidx_v[sl] = srcs_v[sl] + h * N
                return 0

            lax.fori_loop(0, K // 16, gix, 0)
            pltpu.async_copy(xpjr.at[gidx_v], rows_v, sem).wait()

            def chunk(c, _):
                dv = dsts_v[pl.ds(c * 16, 16)]
                av = alph_v[pl.ds(c * 16, 16)]
                for l in range(16):
                    pos = base + c * 16 + l
                    valid = jnp.logical_and(pos >= e0, pos < e1)
                    d_s = dv[l] - lo
                    d_s = jnp.where(valid, d_s, 0)
                    d_s = jnp.clip(d_s, 0, P - 1)
                    a_s = av[l] * valid.astype(jnp.float32)
                    for q in range(8):
                        sl = pl.ds(q * 16, 16)
                        tmp_v[pl.ds(q * 16, 16)] = a_s * rows_v[c * 16 + l, sl]
                    for q in range(8):
                        sl = pl.ds(q * 16, 16)
                        acc[d_s, sl] = acc[d_s, sl] + tmp_v[sl]
                return 0

            lax.fori_loop(0, K // 16, chunk, 0)
            return 0

        lax.fori_loop(0, nb, batch, 0)

        @pl.when(lo + P <= N)
        def _():
            pltpu.sync_copy(acc, out.at[pl.ds(lo, P), h])

        @pl.when(lo + P > N)
        def _():
            pltpu.sync_copy(acc.at[pl.ds(0, N - (NW - 1) * P)],
                            out.at[pl.ds(lo, N - (NW - 1) * P), h])


@jax.jit
def _fold_sc(xpjr, srcs, dsts, alphf, rp):
    mesh = plsc.VectorSubcoreMesh(core_axis_name="c", subcore_axis_name="s")
    fn = pl.kernel(
        _fold_body,
        mesh=mesh,
        out_type=jax.ShapeDtypeStruct((N, HEADS, D), jnp.float32),
        scratch_types=[
            pltpu.VMEM((K,), jnp.int32),
            pltpu.VMEM((K,), jnp.int32),
            pltpu.VMEM((K,), jnp.int32),
            pltpu.VMEM((K,), jnp.float32),
            pltpu.VMEM((K, D), jnp.float32),
            pltpu.VMEM((P, D), jnp.float32),
            pltpu.VMEM((16,), jnp.int32),
            pltpu.VMEM((D,), jnp.float32),
            pltpu.SemaphoreType.DMA,
        ],
    )
    return fn(xpjr, srcs, dsts, alphf, rp)


def kernel(x, edge_index, W_gnn, b_gnn, weight, attention, lin_W, lin_b, s_lin1_W, s_lin2_W, s_lin2_b):
    loop = jnp.arange(N, dtype=edge_index.dtype)
    src = jnp.concatenate([edge_index[0], loop])
    dst = jnp.concatenate([edge_index[1], loop])
    ones = jnp.ones_like(src, jnp.float32)
    deg = jax.ops.segment_sum(ones, dst, num_segments=N)
    dinv = 1.0 / jnp.sqrt(jnp.clip(deg, 1.0, None))
    h = x @ W_gnn
    norm = dinv[src] * dinv[dst]
    x_pool_j = jax.ops.segment_sum(norm[:, None] * h[src], dst, num_segments=N) + b_gnn
    xpj = (x_pool_j @ weight).reshape(N, HEADS, D)
    x_q = x @ lin_W + lin_b
    q_i = jnp.broadcast_to(x_q[dst][:, None, :], (src.shape[0], HEADS, D))
    cat = jnp.concatenate([q_i, xpj[src]], axis=-1)
    logits = jnp.sum(cat * attention, axis=-1)
    logits = jax.nn.leaky_relu(logits, NEG_SLOPE)
    alpha = _segment_softmax(logits, dst, N)

    order = jnp.argsort(dst, stable=True).astype(jnp.int32)
    dsts_s = dst[order].astype(jnp.int32)
    srcs_s = src[order].astype(jnp.int32)
    degp = jax.ops.segment_sum(jnp.ones((EP,), jnp.int32), dst, num_segments=NPAD)
    rp = jnp.concatenate([jnp.zeros((1,), jnp.int32), jnp.cumsum(degp).astype(jnp.int32)])
    rp = jnp.concatenate([rp, jnp.full((RPLEN - rp.shape[0],), EP, jnp.int32)])
    srcs_pad = jnp.concatenate([srcs_s, jnp.zeros((EPAD - EP,), jnp.int32)])
    dsts_pad = jnp.concatenate([dsts_s, jnp.zeros((EPAD - EP,), jnp.int32)])
    alph_s = alpha[order]
    alphf = jnp.concatenate(
        [alph_s.T, jnp.zeros((HEADS, EPAD - EP), jnp.float32)], axis=1
    ).reshape(HEADS * EPAD)
    xpjr = xpj.transpose(1, 0, 2).reshape(HEADS * N, D)
    x_cluster_h = _fold_sc(xpjr, srcs_pad, dsts_pad, alphf, rp)
    x_cluster = jnp.mean(x_cluster_h, axis=1)
    a = x_cluster @ s_lin1_W
    b = x_cluster @ s_lin2_W + s_lin2_b
    ai, aj = a[dst], a[src]
    cos = jnp.sum(ai * aj, axis=-1) / (jnp.linalg.norm(ai, axis=-1) * jnp.linalg.norm(aj, axis=-1) + 1e-8)
    fitness = jax.ops.segment_sum(b[dst] * cos[:, None], dst, num_segments=N)
    score = jax.nn.sigmoid(fitness[:, 0])
    k = int(math.ceil(RATIO * N))
    top_scores, perm = jax.lax.top_k(score, k)
    new_x = x_cluster[perm] * top_scores[:, None]
    return new_x, score, perm
